# baseline (speedup 1.0000x reference)
import functools

import jax
import jax.numpy as jnp
from jax import lax
from jax.experimental import pallas as pl
from jax.experimental.pallas import tpu as pltpu, tpu_sc as plsc


def _make_lookup(V, D, B):
    info = plsc.get_sparse_core_info()
    NC, NS, L = info.num_cores, info.num_subcores, info.num_lanes
    NW = NC * NS
    assert B % (8 * NW) == 0
    b_per_w = B // NW
    n_desc = b_per_w // 128
    mesh = plsc.VectorSubcoreMesh(core_axis_name="c", subcore_axis_name="s")

    @functools.partial(
        pl.kernel,
        mesh=mesh,
        out_type=jax.ShapeDtypeStruct((B, D), jnp.float32),
        scratch_types=[
            pltpu.VMEM((n_desc, 128), jnp.int32),
            pltpu.VMEM((b_per_w, D), jnp.float32),
            pltpu.SemaphoreType.DMA,
        ],
        compiler_params=pltpu.CompilerParams(use_tc_tiling_on_sc=False),
    )
    def k(table_hbm, idx_hbm, out_hbm, pidx_v, rows_v, sem):
        wid = lax.axis_index("s") * NC + lax.axis_index("c")
        base = wid * b_per_w
        for j in range(n_desc):
            pltpu.sync_copy(idx_hbm.at[pl.ds(base + j * 128, 128)], pidx_v.at[j])
        for j in range(n_desc):
            pltpu.async_copy(
                table_hbm.at[pidx_v.at[j]], rows_v.at[pl.ds(j * 128, 128)], sem
            )
        pltpu.make_async_copy(table_hbm.at[pl.ds(0, b_per_w)], rows_v, sem).wait()
        pltpu.sync_copy(rows_v, out_hbm.at[pl.ds(base, b_per_w)])

    return k


def kernel(labels, embed_table):
    B = labels.shape[0]
    V, D = embed_table.shape
    lookup = _make_lookup(V, D, B)
    return lookup(embed_table, labels.astype(jnp.int32))


# speedup vs baseline: 1.6556x; 1.6556x over previous
import functools

import jax
import jax.numpy as jnp
from jax import lax
from jax.experimental import pallas as pl
from jax.experimental.pallas import tpu as pltpu, tpu_sc as plsc


def _make_lookup(V, D, B):
    info = plsc.get_sparse_core_info()
    NC, NS, L = info.num_cores, info.num_subcores, info.num_lanes
    NW = NC * NS
    assert B % (8 * NW) == 0
    b_per_w = B // NW
    NSEM = 8
    mesh = plsc.VectorSubcoreMesh(core_axis_name="c", subcore_axis_name="s")

    @functools.partial(
        pl.kernel,
        mesh=mesh,
        out_type=jax.ShapeDtypeStruct((B, D), jnp.float32),
        scratch_types=[
            pltpu.VMEM((b_per_w,), jnp.int32),
            pltpu.VMEM((b_per_w, D), jnp.float32),
            [pltpu.SemaphoreType.DMA] * NSEM,
        ],
    )
    def k(table_hbm, idx_hbm, out_hbm, idx_v, rows_v, sems):
        wid = lax.axis_index("s") * NC + lax.axis_index("c")
        base = wid * b_per_w
        pltpu.sync_copy(idx_hbm.at[pl.ds(base, b_per_w)], idx_v)

        @plsc.parallel_loop(0, b_per_w // L, unroll=2)
        def fire(c):
            iv = idx_v[pl.ds(c * L, L)]
            for u in range(L):
                r = c * L + u
                pltpu.async_copy(
                    table_hbm.at[pl.ds(iv[u], 1)],
                    rows_v.at[pl.ds(r, 1)],
                    sems[u % NSEM],
                )

        for s in range(NSEM):
            pltpu.make_async_copy(
                table_hbm.at[pl.ds(0, b_per_w // NSEM)],
                rows_v.at[pl.ds(s * (b_per_w // NSEM), b_per_w // NSEM)],
                sems[s],
            ).wait()
        pltpu.sync_copy(rows_v, out_hbm.at[pl.ds(base, b_per_w)])

    return k


def kernel(labels, embed_table):
    B = labels.shape[0]
    V, D = embed_table.shape
    lookup = _make_lookup(V, D, B)
    return lookup(embed_table, labels.astype(jnp.int32))


# speedup vs baseline: 1.6582x; 1.0015x over previous
import functools

import jax
import jax.numpy as jnp
from jax import lax
from jax.experimental import pallas as pl
from jax.experimental.pallas import tpu as pltpu, tpu_sc as plsc


def _make_lookup(V, D, B):
    info = plsc.get_sparse_core_info()
    NC, NS, L = info.num_cores, info.num_subcores, info.num_lanes
    NW = NC * NS
    assert B % (8 * NW) == 0
    b_per_w = B // NW
    mesh = plsc.VectorSubcoreMesh(core_axis_name="c", subcore_axis_name="s")

    @functools.partial(
        pl.kernel,
        mesh=mesh,
        out_type=jax.ShapeDtypeStruct((B, D), jnp.float32),
        scratch_types=[
            pltpu.VMEM((b_per_w,), jnp.int32),
            pltpu.VMEM((b_per_w, D), jnp.float32),
            pltpu.SemaphoreType.DMA,
        ],
    )
    def k(table_hbm, idx_hbm, out_hbm, idx_v, rows_v, sem):
        wid = lax.axis_index("s") * NC + lax.axis_index("c")
        base = wid * b_per_w
        pltpu.sync_copy(idx_hbm.at[pl.ds(base, b_per_w)], idx_v)

        @plsc.parallel_loop(0, b_per_w // L, unroll=2)
        def fire(c):
            iv = idx_v[pl.ds(c * L, L)]
            for u in range(L):
                r = c * L + u
                pltpu.async_copy(
                    table_hbm.at[iv[u]], rows_v.at[r], sem
                )

        pltpu.make_async_copy(
            table_hbm.at[pl.ds(0, b_per_w)], rows_v, sem
        ).wait()
        pltpu.sync_copy(rows_v, out_hbm.at[pl.ds(base, b_per_w)])

    return k


def kernel(labels, embed_table):
    B = labels.shape[0]
    V, D = embed_table.shape
    lookup = _make_lookup(V, D, B)
    return lookup(embed_table, labels.astype(jnp.int32))
